# baseline (speedup 1.0000x reference)
import functools

import jax
import jax.numpy as jnp
from jax import lax
from jax.experimental import pallas as pl
from jax.experimental.pallas import tpu as pltpu
from jax.experimental.pallas import tpu_sc as plsc

_B = 16384
_D = 32

_info = plsc.get_sparse_core_info()
_NC = _info.num_cores
_NS = _info.num_subcores
_NW = _NC * _NS
_BPW = _B // _NW

_mesh = plsc.VectorSubcoreMesh(core_axis_name="c", subcore_axis_name="s")


@functools.partial(
    pl.kernel,
    mesh=_mesh,
    out_type=jax.ShapeDtypeStruct((4 * _B, _D), jnp.float32),
    compiler_params=pltpu.CompilerParams(use_tc_tiling_on_sc=False),
    scratch_types=[
        pltpu.VMEM((_BPW,), jnp.int32),
        pltpu.VMEM((_BPW,), jnp.int32),
        pltpu.VMEM((_BPW,), jnp.int32),
        pltpu.VMEM((_BPW,), jnp.int32),
        pltpu.VMEM((_BPW, _D), jnp.float32),
        pltpu.VMEM((_BPW, _D), jnp.float32),
        pltpu.VMEM((_BPW, _D), jnp.float32),
        pltpu.VMEM((_BPW, _D), jnp.float32),
        pltpu.SemaphoreType.DMA,
        pltpu.SemaphoreType.DMA,
        pltpu.SemaphoreType.DMA,
        pltpu.SemaphoreType.DMA,
        pltpu.SemaphoreType.DMA,
    ],
)
def _emb_kernel(cat_i, sub_i, elem_i, evt_i,
                cat_t, sub_t, elem_t, evt_t, out,
                i0, i1, i2, i3, r0, r1, r2, r3,
                g0, g1, g2, g3, ws):
    wid = lax.axis_index("s") * _NC + lax.axis_index("c")
    base = wid * _BPW
    pltpu.sync_copy(cat_i.at[pl.ds(base, _BPW)], i0)
    pltpu.sync_copy(sub_i.at[pl.ds(base, _BPW)], i1)
    pltpu.sync_copy(elem_i.at[pl.ds(base, _BPW)], i2)
    pltpu.sync_copy(evt_i.at[pl.ds(base, _BPW)], i3)
    c0 = pltpu.async_copy(cat_t.at[i0], r0, g0)
    c1 = pltpu.async_copy(sub_t.at[i1], r1, g1)
    c2 = pltpu.async_copy(elem_t.at[i2], r2, g2)
    c3 = pltpu.async_copy(evt_t.at[i3], r3, g3)
    c0.wait()
    w0 = pltpu.async_copy(r0, out.at[pl.ds(0 * _B + base, _BPW)], ws)
    c1.wait()
    w1 = pltpu.async_copy(r1, out.at[pl.ds(1 * _B + base, _BPW)], ws)
    c2.wait()
    w2 = pltpu.async_copy(r2, out.at[pl.ds(2 * _B + base, _BPW)], ws)
    c3.wait()
    w3 = pltpu.async_copy(r3, out.at[pl.ds(3 * _B + base, _BPW)], ws)
    w0.wait()
    w1.wait()
    w2.wait()
    w3.wait()


def kernel(categories, sub_categories, elements, event_types,
           category_table, sub_category_table, element_table,
           event_type_table):
    cat_i = jnp.asarray(categories, jnp.int32)
    sub_i = jnp.asarray(sub_categories, jnp.int32)
    elem_i = jnp.asarray(elements, jnp.int32)
    evt_i = jnp.asarray(event_types, jnp.int32)
    return _emb_kernel(cat_i, sub_i, elem_i, evt_i,
                       category_table, sub_category_table,
                       element_table, event_type_table)
